# baseline (speedup 1.0000x reference)
import functools

import jax
import jax.numpy as jnp
from jax import lax
from jax.experimental import pallas as pl
from jax.experimental.pallas import tpu as pltpu
from jax.experimental.pallas import tpu_sc as plsc

NC, NS, L = 2, 16, 16
B = 16
K = 500
KP = 512
NCHUNK = KP // L
HW = 128 * 128
NACC = 4


def _tec_body(tag1_hbm, tag2_hbm, ind1_hbm, ind2_hbm, mask_hbm, out_hbm,
              tagrow, ind_v, mask_v, tmine, tother, mcomp, row_v, shared):
    c = lax.axis_index("c")
    s = lax.axis_index("s")
    b = c * (B // NC) + (s >> 1)
    h = s & 1
    iota = lax.iota(jnp.int32, L)

    bk = b * K
    off = jnp.minimum(bk & ~7, B * K - KP)
    off = pl.multiple_of(off, 8)
    shift = bk - off
    pltpu.sync_copy(mask_hbm.at[pl.ds(off, KP)], mask_v)

    @pl.when(h == 0)
    def _():
        pltpu.sync_copy(ind1_hbm.at[pl.ds(off, KP)], ind_v)
        pltpu.sync_copy(tag1_hbm.at[b], tagrow)

    @pl.when(h == 1)
    def _():
        pltpu.sync_copy(ind2_hbm.at[pl.ds(off, KP)], ind_v)
        pltpu.sync_copy(tag2_hbm.at[b], tagrow)

    kmax = shift + (K - 1)
    for jc in range(NCHUNK):
        kidx = jnp.full((L,), shift + jc * L, jnp.int32) + iota
        if (jc + 1) * L > K:
            kidx = jnp.minimum(kidx, kmax)
        ii = plsc.load_gather(ind_v, [kidx])
        tmine[pl.ds(jc * L, L)] = plsc.load_gather(tagrow, [ii])

    pltpu.sync_copy(tmine, shared.at[s])
    plsc.subcore_barrier()
    pltpu.sync_copy(shared.at[s ^ 1], tother)

    for jc in range(NCHUNK):
        mcomp[pl.ds(jc * L, L)] = (
            2.0e6 + 2.0 * (jc * L + iota).astype(jnp.float32))
    offset = jnp.int32(0)
    psum = jnp.zeros((L,), jnp.float32)
    for jc in range(NCHUNK):
        sl = pl.ds(jc * L, L)
        kidx = jnp.full((L,), shift + jc * L, jnp.int32) + iota
        if (jc + 1) * L > K:
            kidx = jnp.minimum(kidx, kmax)
        m = plsc.load_gather(mask_v, [kidx])
        mb = m > 0
        if (jc + 1) * L > K:
            mb = mb & (jc * L + iota < K)
            m = jnp.where(mb, m, 0)
        a = tmine[sl]
        bb = tother[sl]
        mean = (a + bb) * 0.5
        d = a - bb
        psum = psum + jnp.where(mb, d * d, 0.0)
        pos = jnp.maximum(offset + lax.cumsum(m, axis=0) - 1, 0)
        plsc.store_scatter(mcomp, [pos], mean, mask=mb)
        offset = offset + jnp.sum(m)
    n = offset
    pullsq = jnp.sum(psum)

    nb = (n + L - 1) >> 4
    my_blocks = (nb - h + 1) >> 1
    zero = jnp.zeros((L,), jnp.float32)

    def outer(t, accs):
        rb = 2 * t + h
        base = rb * L
        basev = jnp.full((L,), base, jnp.int32)
        riv = plsc.load_gather(mcomp, [basev + iota])
        valid = (basev + iota) < n
        mis = [plsc.load_gather(mcomp, [jnp.full((L,), base + l, jnp.int32)])
               for l in range(L)]
        accl = list(accs)
        for l in range(L):
            r = jnp.maximum(1.0 - jnp.abs(riv - mis[l]), 0.0)
            accl[l % NACC] = accl[l % NACC] + jnp.where(
                (iota >= l) & valid, r, 0.0)

        def inner(jc, accs2):
            mjv = plsc.load_gather(
                mcomp, [jnp.full((L,), jc * L, jnp.int32) + iota])
            a2 = list(accs2)
            for l in range(L):
                r = jnp.maximum(1.0 - jnp.abs(mjv - mis[l]), 0.0)
                a2[l % NACC] = a2[l % NACC] + r
            return tuple(a2)

        return lax.fori_loop(rb + 1, nb, inner, tuple(accl))

    accs = lax.fori_loop(0, my_blocks, outer, (zero,) * NACC)
    Sp = jnp.sum(sum(accs[1:], accs[0]))

    h0 = h == 0
    nfv = jnp.full((L,), n.astype(jnp.float32))
    pullv = jnp.full((L,), jnp.where(h0, pullsq, 0.0)) / (2.0 * (nfv + 1e-4))
    corr = jnp.where(h0, nfv + nfv * nfv / (nfv + 1e-4), 0.0)
    pushv = (2.0 * jnp.full((L,), Sp) - corr) / ((nfv - 1.0) * nfv + 1e-4)
    lane = lax.iota(jnp.int32, L)
    row_v[...] = jnp.where(lane == 0, pullv,
                           jnp.where(lane == 1, pushv, 0.0))
    pltpu.sync_copy(row_v, out_hbm.at[c * NS + s])


@functools.partial(
    pl.kernel,
    out_type=jax.ShapeDtypeStruct((NC * NS, L), jnp.float32),
    mesh=plsc.VectorSubcoreMesh(core_axis_name="c", subcore_axis_name="s"),
    compiler_params=pltpu.CompilerParams(needs_layout_passes=False),
    scratch_types=[
        pltpu.VMEM((HW,), jnp.float32),
        pltpu.VMEM((KP,), jnp.int32),
        pltpu.VMEM((KP,), jnp.int32),
        pltpu.VMEM((KP,), jnp.float32),
        pltpu.VMEM((KP,), jnp.float32),
        pltpu.VMEM((KP,), jnp.float32),
        pltpu.VMEM((L,), jnp.float32),
        pltpu.VMEM_SHARED((NS, KP), jnp.float32),
    ],
)
def _tag_loss_sc(tag1_hbm, tag2_hbm, ind1_hbm, ind2_hbm, mask_hbm, out_hbm,
                 *scratch):
    _tec_body(tag1_hbm, tag2_hbm, ind1_hbm, ind2_hbm, mask_hbm, out_hbm,
              *scratch)


@jax.jit
def kernel(tag1, tag2, ind1, ind2, mask):
    tag1f = tag1.reshape(B, HW)
    tag2f = tag2.reshape(B, HW)
    ind1f = ind1.astype(jnp.int32).reshape(B * K)
    ind2f = ind2.astype(jnp.int32).reshape(B * K)
    maskf = mask.astype(jnp.int32).reshape(B * K)
    out = _tag_loss_sc(tag1f, tag2f, ind1f, ind2f, maskf)
    return (out[:, 0].sum(), out[:, 1].sum())


# speedup vs baseline: 1.0438x; 1.0438x over previous
import functools

import jax
import jax.numpy as jnp
from jax import lax
from jax.experimental import pallas as pl
from jax.experimental.pallas import tpu as pltpu
from jax.experimental.pallas import tpu_sc as plsc

NC, NS, L = 2, 16, 16
B = 16
K = 500
KP = 512
NCHUNK = KP // L
HW = 128 * 128
NACC = 4


def _tec_body(tag1_hbm, tag2_hbm, cat_hbm, out_hbm,
              tag_a, tag_b, ind_v, mask_v, tmine, tother, mcomp, row_v):
    c = lax.axis_index("c")
    s = lax.axis_index("s")
    b = c * (B // NC) + (s >> 1)
    h = s & 1
    iota = lax.iota(jnp.int32, L)

    bk = b * K
    woff = jnp.minimum(bk & ~7, B * K - KP)
    shift = bk - woff
    mask_off = pl.multiple_of(2 * (B * K) + woff, 8)
    pltpu.sync_copy(cat_hbm.at[pl.ds(mask_off, KP)], mask_v)
    pltpu.sync_copy(tag1_hbm.at[b], tag_a)
    pltpu.sync_copy(tag2_hbm.at[b], tag_b)

    kmax = shift + (K - 1)

    def gather_tag(tagref, ind_off, dst):
        pltpu.sync_copy(cat_hbm.at[pl.ds(ind_off, KP)], ind_v)
        for jc in range(NCHUNK):
            kidx = jnp.full((L,), shift + jc * L, jnp.int32) + iota
            if (jc + 1) * L > K:
                kidx = jnp.minimum(kidx, kmax)
            ii = plsc.load_gather(ind_v, [kidx])
            dst[pl.ds(jc * L, L)] = plsc.load_gather(
                tagref, [ii >> 7, ii & 127])

    gather_tag(tag_a, pl.multiple_of(woff, 8), tmine)
    gather_tag(tag_b, pl.multiple_of(B * K + woff, 8), tother)

    for jc in range(NCHUNK):
        mcomp[pl.ds(jc * L, L)] = (
            2.0e6 + 2.0 * (jc * L + iota).astype(jnp.float32))
    offset = jnp.int32(0)
    psum = jnp.zeros((L,), jnp.float32)
    for jc in range(NCHUNK):
        sl = pl.ds(jc * L, L)
        kidx = jnp.full((L,), shift + jc * L, jnp.int32) + iota
        if (jc + 1) * L > K:
            kidx = jnp.minimum(kidx, kmax)
        m = plsc.load_gather(mask_v, [kidx])
        mb = m > 0
        if (jc + 1) * L > K:
            mb = mb & (jc * L + iota < K)
            m = jnp.where(mb, m, 0)
        a = tmine[sl]
        bb = tother[sl]
        mean = (a + bb) * 0.5
        d = a - bb
        psum = psum + jnp.where(mb, d * d, 0.0)
        pos = jnp.maximum(offset + lax.cumsum(m, axis=0) - 1, 0)
        plsc.store_scatter(mcomp, [pos], mean, mask=mb)
        offset = offset + jnp.sum(m)
    n = offset
    pullsq = jnp.sum(psum)

    nb = (n + L - 1) >> 4
    my_blocks = (nb - h + 1) >> 1
    zero = jnp.zeros((L,), jnp.float32)

    def outer(t, accs):
        rb = 2 * t + h
        base = rb * L
        basev = jnp.full((L,), base, jnp.int32)
        riv = plsc.load_gather(mcomp, [basev + iota])
        valid = (basev + iota) < n
        mis = [plsc.load_gather(mcomp, [jnp.full((L,), base + l, jnp.int32)])
               for l in range(L)]
        accl = list(accs)
        for l in range(L):
            r = jnp.maximum(1.0 - jnp.abs(riv - mis[l]), 0.0)
            accl[l % NACC] = accl[l % NACC] + jnp.where(
                (iota >= l) & valid, r, 0.0)

        def inner(jc, accs2):
            mjv = plsc.load_gather(
                mcomp, [jnp.full((L,), jc * L, jnp.int32) + iota])
            a2 = list(accs2)
            for l in range(L):
                r = jnp.maximum(1.0 - jnp.abs(mjv - mis[l]), 0.0)
                a2[l % NACC] = a2[l % NACC] + r
            return tuple(a2)

        return lax.fori_loop(rb + 1, nb, inner, tuple(accl))

    accs = lax.fori_loop(0, my_blocks, outer, (zero,) * NACC)
    Sp = jnp.sum(sum(accs[1:], accs[0]))

    h0 = h == 0
    nfv = jnp.full((L,), n.astype(jnp.float32))
    pullv = jnp.full((L,), jnp.where(h0, pullsq, 0.0)) / (2.0 * (nfv + 1e-4))
    corr = jnp.where(h0, nfv + nfv * nfv / (nfv + 1e-4), 0.0)
    pushv = (2.0 * jnp.full((L,), Sp) - corr) / ((nfv - 1.0) * nfv + 1e-4)
    lane = lax.iota(jnp.int32, L)
    row_v[...] = jnp.where(lane == 0, pullv,
                           jnp.where(lane == 1, pushv, 0.0))
    pltpu.sync_copy(row_v, out_hbm.at[c * NS + s])


@functools.partial(
    pl.kernel,
    out_type=jax.ShapeDtypeStruct((NC * NS, L), jnp.float32),
    mesh=plsc.VectorSubcoreMesh(core_axis_name="c", subcore_axis_name="s"),
    compiler_params=pltpu.CompilerParams(needs_layout_passes=False),
    scratch_types=[
        pltpu.VMEM((128, 128), jnp.float32),
        pltpu.VMEM((128, 128), jnp.float32),
        pltpu.VMEM((KP,), jnp.int32),
        pltpu.VMEM((KP,), jnp.int32),
        pltpu.VMEM((KP,), jnp.float32),
        pltpu.VMEM((KP,), jnp.float32),
        pltpu.VMEM((KP,), jnp.float32),
        pltpu.VMEM((L,), jnp.float32),
    ],
)
def _tag_loss_sc(tag1_hbm, tag2_hbm, cat_hbm, out_hbm, *scratch):
    _tec_body(tag1_hbm, tag2_hbm, cat_hbm, out_hbm, *scratch)


@jax.jit
def kernel(tag1, tag2, ind1, ind2, mask):
    tag1s = tag1.reshape(B, 128, 128)
    tag2s = tag2.reshape(B, 128, 128)
    cat = jnp.concatenate([
        ind1.astype(jnp.int32).reshape(B * K),
        ind2.astype(jnp.int32).reshape(B * K),
        mask.astype(jnp.int32).reshape(B * K),
    ])
    out = _tag_loss_sc(tag1s, tag2s, cat)
    return (out[:, 0].sum(), out[:, 1].sum())


# speedup vs baseline: 1.1216x; 1.0745x over previous
import functools

import jax
import jax.numpy as jnp
from jax import lax
from jax.experimental import pallas as pl
from jax.experimental.pallas import tpu as pltpu
from jax.experimental.pallas import tpu_sc as plsc

NC, NS, L = 2, 16, 16
B = 16
K = 500
KP = 512
NCHUNK = KP // L
HW = 128 * 128
NACC = 4


def _tec_body(tag1_hbm, tag2_hbm, cat_hbm, out_hbm,
              tag_a, tag_b, ind_v, ind2_v, mask_v, tmine, tother, mcomp,
              row_v, sem):
    c = lax.axis_index("c")
    s = lax.axis_index("s")
    b = c * (B // NC) + (s >> 1)
    h = s & 1
    iota = lax.iota(jnp.int32, L)

    bk = b * K
    woff = jnp.minimum(bk & ~7, B * K - KP)
    shift = bk - woff
    mask_off = pl.multiple_of(2 * (B * K) + woff, 8)
    cps = [
        pltpu.async_copy(cat_hbm.at[pl.ds(mask_off, KP)], mask_v, sem),
        pltpu.async_copy(
            cat_hbm.at[pl.ds(pl.multiple_of(woff, 8), KP)], ind_v, sem),
        pltpu.async_copy(
            cat_hbm.at[pl.ds(pl.multiple_of(B * K + woff, 8), KP)],
            ind2_v, sem),
        pltpu.async_copy(tag1_hbm.at[b, 0], tag_a, sem),
        pltpu.async_copy(tag2_hbm.at[b, 0], tag_b, sem),
    ]
    for cp in cps:
        cp.wait()

    kmax = shift + (K - 1)

    def gather_tag(tagref, idx_ref, dst):
        for jc in range(NCHUNK):
            kidx = jnp.full((L,), shift + jc * L, jnp.int32) + iota
            if (jc + 1) * L > K:
                kidx = jnp.minimum(kidx, kmax)
            ii = plsc.load_gather(idx_ref, [kidx])
            dst[pl.ds(jc * L, L)] = plsc.load_gather(
                tagref, [ii >> 7, ii & 127])

    gather_tag(tag_a, ind_v, tmine)
    gather_tag(tag_b, ind2_v, tother)

    for jc in range(NCHUNK):
        mcomp[pl.ds(jc * L, L)] = (
            2.0e6 + 2.0 * (jc * L + iota).astype(jnp.float32))
    offset = jnp.int32(0)
    psum = jnp.zeros((L,), jnp.float32)
    for jc in range(NCHUNK):
        sl = pl.ds(jc * L, L)
        kidx = jnp.full((L,), shift + jc * L, jnp.int32) + iota
        if (jc + 1) * L > K:
            kidx = jnp.minimum(kidx, kmax)
        m = plsc.load_gather(mask_v, [kidx])
        mb = m > 0
        if (jc + 1) * L > K:
            mb = mb & (jc * L + iota < K)
            m = jnp.where(mb, m, 0)
        a = tmine[sl]
        bb = tother[sl]
        mean = (a + bb) * 0.5
        d = a - bb
        psum = psum + jnp.where(mb, d * d, 0.0)
        pos = jnp.maximum(offset + lax.cumsum(m, axis=0) - 1, 0)
        plsc.store_scatter(mcomp, [pos], mean, mask=mb)
        offset = offset + jnp.sum(m)
    n = offset
    pullsq = jnp.sum(psum)

    nb = (n + L - 1) >> 4
    my_blocks = (nb - h + 1) >> 1
    zero = jnp.zeros((L,), jnp.float32)

    def outer(t, accs):
        rb = 2 * t + h
        base = rb * L
        basev = jnp.full((L,), base, jnp.int32)
        riv = plsc.load_gather(mcomp, [basev + iota])
        valid = (basev + iota) < n
        mis = [plsc.load_gather(mcomp, [jnp.full((L,), base + l, jnp.int32)])
               for l in range(L)]
        accl = list(accs)
        for l in range(L):
            r = jnp.maximum(1.0 - jnp.abs(riv - mis[l]), 0.0)
            accl[l % NACC] = accl[l % NACC] + jnp.where(
                (iota >= l) & valid, r, 0.0)

        def inner(jc, accs2):
            mjv = plsc.load_gather(
                mcomp, [jnp.full((L,), jc * L, jnp.int32) + iota])
            a2 = list(accs2)
            for l in range(L):
                r = jnp.maximum(1.0 - jnp.abs(mjv - mis[l]), 0.0)
                a2[l % NACC] = a2[l % NACC] + r
            return tuple(a2)

        return lax.fori_loop(rb + 1, nb, inner, tuple(accl))

    accs = lax.fori_loop(0, my_blocks, outer, (zero,) * NACC)
    Sp = jnp.sum(sum(accs[1:], accs[0]))

    h0 = h == 0
    nfv = jnp.full((L,), n.astype(jnp.float32))
    pullv = jnp.full((L,), jnp.where(h0, pullsq, 0.0)) / (2.0 * (nfv + 1e-4))
    corr = jnp.where(h0, nfv + nfv * nfv / (nfv + 1e-4), 0.0)
    pushv = (2.0 * jnp.full((L,), Sp) - corr) / ((nfv - 1.0) * nfv + 1e-4)
    lane = lax.iota(jnp.int32, L)
    row_v[...] = jnp.where(lane == 0, pullv,
                           jnp.where(lane == 1, pushv, 0.0))
    pltpu.sync_copy(row_v, out_hbm.at[c * NS + s])


@functools.partial(
    pl.kernel,
    out_type=jax.ShapeDtypeStruct((NC * NS, L), jnp.float32),
    mesh=plsc.VectorSubcoreMesh(core_axis_name="c", subcore_axis_name="s"),
    compiler_params=pltpu.CompilerParams(needs_layout_passes=False),
    scratch_types=[
        pltpu.VMEM((128, 128), jnp.float32),
        pltpu.VMEM((128, 128), jnp.float32),
        pltpu.VMEM((KP,), jnp.int32),
        pltpu.VMEM((KP,), jnp.int32),
        pltpu.VMEM((KP,), jnp.int32),
        pltpu.VMEM((KP,), jnp.float32),
        pltpu.VMEM((KP,), jnp.float32),
        pltpu.VMEM((KP,), jnp.float32),
        pltpu.VMEM((L,), jnp.float32),
        pltpu.SemaphoreType.DMA,
    ],
)
def _tag_loss_sc(tag1_hbm, tag2_hbm, cat_hbm, out_hbm, *scratch):
    _tec_body(tag1_hbm, tag2_hbm, cat_hbm, out_hbm, *scratch)


@jax.jit
def kernel(tag1, tag2, ind1, ind2, mask):
    cat = jnp.concatenate([
        ind1.astype(jnp.int32).reshape(B * K),
        ind2.astype(jnp.int32).reshape(B * K),
        mask.astype(jnp.int32).reshape(B * K),
    ])
    out = _tag_loss_sc(tag1, tag2, cat)
    return (out[:, 0].sum(), out[:, 1].sum())


# speedup vs baseline: 1.1478x; 1.0234x over previous
import functools

import jax
import jax.numpy as jnp
from jax import lax
from jax.experimental import pallas as pl
from jax.experimental.pallas import tpu as pltpu
from jax.experimental.pallas import tpu_sc as plsc

NC, NS, L = 2, 16, 16
B = 16
K = 500
KP = 512
NCHUNK = KP // L
HW = 128 * 128
NACC = 4


def _tec_body(tag1_hbm, tag2_hbm, cat_hbm, out_hbm,
              tag_a, tag_b, ind_v, ind2_v, mask_v, posv, mfv, mcomp,
              row_v, sem, sem2):
    c = lax.axis_index("c")
    s = lax.axis_index("s")
    b = c * (B // NC) + (s >> 1)
    h = s & 1
    iota = lax.iota(jnp.int32, L)

    bk = b * K
    woff = jnp.minimum(bk & ~7, B * K - KP)
    shift = bk - woff
    mask_off = pl.multiple_of(2 * (B * K) + woff, 8)
    cps2 = [
        pltpu.async_copy(tag1_hbm.at[b, 0], tag_a, sem2),
        pltpu.async_copy(tag2_hbm.at[b, 0], tag_b, sem2),
    ]
    cps = [
        pltpu.async_copy(cat_hbm.at[pl.ds(mask_off, KP)], mask_v, sem),
        pltpu.async_copy(
            cat_hbm.at[pl.ds(pl.multiple_of(woff, 8), KP)], ind_v, sem),
        pltpu.async_copy(
            cat_hbm.at[pl.ds(pl.multiple_of(B * K + woff, 8), KP)],
            ind2_v, sem),
    ]
    for cp in cps:
        cp.wait()

    kmax = shift + (K - 1)

    for jc in range(NCHUNK):
        mcomp[pl.ds(jc * L, L)] = (
            2.0e6 + 2.0 * (jc * L + iota).astype(jnp.float32))
    offset = jnp.int32(0)
    for jc in range(NCHUNK):
        sl = pl.ds(jc * L, L)
        kidx = jnp.full((L,), shift + jc * L, jnp.int32) + iota
        if (jc + 1) * L > K:
            kidx = jnp.minimum(kidx, kmax)
        m = plsc.load_gather(mask_v, [kidx])
        mb = m > 0
        if (jc + 1) * L > K:
            mb = mb & (jc * L + iota < K)
            m = jnp.where(mb, m, 0)
        mfv[sl] = m.astype(jnp.float32)
        pos = offset + lax.cumsum(m, axis=0) - 1
        posv[sl] = jnp.where(mb, pos, KP + iota)
        offset = offset + jnp.sum(m)
    n = offset

    for cp in cps2:
        cp.wait()
    psum = jnp.zeros((L,), jnp.float32)
    for jc in range(NCHUNK):
        sl = pl.ds(jc * L, L)
        kidx = jnp.full((L,), shift + jc * L, jnp.int32) + iota
        if (jc + 1) * L > K:
            kidx = jnp.minimum(kidx, kmax)
        ii0 = plsc.load_gather(ind_v, [kidx])
        ii1 = plsc.load_gather(ind2_v, [kidx])
        t0 = plsc.load_gather(tag_a, [ii0 >> 7, ii0 & 127])
        t1 = plsc.load_gather(tag_b, [ii1 >> 7, ii1 & 127])
        mean = (t0 + t1) * 0.5
        d = t0 - t1
        psum = psum + mfv[sl] * (d * d)
        plsc.store_scatter(mcomp, [posv[sl]], mean)
    pullsq = jnp.sum(psum)

    nb = (n + L - 1) >> 4
    my_blocks = (nb - h + 1) >> 1
    zero = jnp.zeros((L,), jnp.float32)

    def outer(t, accs):
        rb = 2 * t + h
        base = rb * L
        basev = jnp.full((L,), base, jnp.int32)
        riv = plsc.load_gather(mcomp, [basev + iota])
        valid = (basev + iota) < n
        mis = [plsc.load_gather(mcomp, [jnp.full((L,), base + l, jnp.int32)])
               for l in range(L)]
        accl = list(accs)
        for l in range(L):
            r = jnp.maximum(1.0 - jnp.abs(riv - mis[l]), 0.0)
            accl[l % NACC] = accl[l % NACC] + jnp.where(
                (iota >= l) & valid, r, 0.0)

        def inner(jc, accs2):
            mjv = plsc.load_gather(
                mcomp, [jnp.full((L,), jc * L, jnp.int32) + iota])
            a2 = list(accs2)
            for l in range(L):
                r = jnp.maximum(1.0 - jnp.abs(mjv - mis[l]), 0.0)
                a2[l % NACC] = a2[l % NACC] + r
            return tuple(a2)

        return lax.fori_loop(rb + 1, nb, inner, tuple(accl))

    accs = lax.fori_loop(0, my_blocks, outer, (zero,) * NACC)
    Sp = jnp.sum(sum(accs[1:], accs[0]))

    h0 = h == 0
    nfv = jnp.full((L,), n.astype(jnp.float32))
    pullv = jnp.full((L,), jnp.where(h0, pullsq, 0.0)) / (2.0 * (nfv + 1e-4))
    corr = jnp.where(h0, nfv + nfv * nfv / (nfv + 1e-4), 0.0)
    pushv = (2.0 * jnp.full((L,), Sp) - corr) / ((nfv - 1.0) * nfv + 1e-4)
    lane = lax.iota(jnp.int32, L)
    row_v[...] = jnp.where(lane == 0, pullv,
                           jnp.where(lane == 1, pushv, 0.0))
    pltpu.sync_copy(row_v, out_hbm.at[c * NS + s])


@functools.partial(
    pl.kernel,
    out_type=jax.ShapeDtypeStruct((NC * NS, L), jnp.float32),
    mesh=plsc.VectorSubcoreMesh(core_axis_name="c", subcore_axis_name="s"),
    compiler_params=pltpu.CompilerParams(needs_layout_passes=False),
    scratch_types=[
        pltpu.VMEM((128, 128), jnp.float32),
        pltpu.VMEM((128, 128), jnp.float32),
        pltpu.VMEM((KP,), jnp.int32),
        pltpu.VMEM((KP,), jnp.int32),
        pltpu.VMEM((KP,), jnp.int32),
        pltpu.VMEM((KP,), jnp.int32),
        pltpu.VMEM((KP,), jnp.float32),
        pltpu.VMEM((KP + L,), jnp.float32),
        pltpu.VMEM((L,), jnp.float32),
        pltpu.SemaphoreType.DMA,
        pltpu.SemaphoreType.DMA,
    ],
)
def _tag_loss_sc(tag1_hbm, tag2_hbm, cat_hbm, out_hbm, *scratch):
    _tec_body(tag1_hbm, tag2_hbm, cat_hbm, out_hbm, *scratch)


@jax.jit
def kernel(tag1, tag2, ind1, ind2, mask):
    cat = jnp.concatenate([
        ind1.astype(jnp.int32).reshape(B * K),
        ind2.astype(jnp.int32).reshape(B * K),
        mask.astype(jnp.int32).reshape(B * K),
    ])
    out = _tag_loss_sc(tag1, tag2, cat)
    return (out[:, 0].sum(), out[:, 1].sum())


# speedup vs baseline: 1.1543x; 1.0056x over previous
import functools

import jax
import jax.numpy as jnp
from jax import lax
from jax.experimental import pallas as pl
from jax.experimental.pallas import tpu as pltpu
from jax.experimental.pallas import tpu_sc as plsc

NC, NS, L = 2, 16, 16
B = 16
K = 500
KP = 512
NCHUNK = KP // L
HW = 128 * 128
NACC = 4


def _tec_body(tag1_hbm, tag2_hbm, cat_hbm, out_hbm,
              tag_a, tag_b, ind_v, ind2_v, mask_v, posv, mfv, mcomp,
              row_v, sem, sem2):
    c = lax.axis_index("c")
    s = lax.axis_index("s")
    b = c * (B // NC) + (s >> 1)
    h = s & 1
    iota = lax.iota(jnp.int32, L)

    bk = b * K
    woff = jnp.minimum(bk & ~7, B * K - KP)
    shift = bk - woff
    mask_off = pl.multiple_of(2 * (B * K) + woff, 8)
    cps2 = [
        pltpu.async_copy(tag1_hbm.at[b, 0], tag_a, sem2),
        pltpu.async_copy(tag2_hbm.at[b, 0], tag_b, sem2),
    ]
    cps = [
        pltpu.async_copy(cat_hbm.at[pl.ds(mask_off, KP)], mask_v, sem),
        pltpu.async_copy(
            cat_hbm.at[pl.ds(pl.multiple_of(woff, 8), KP)], ind_v, sem),
        pltpu.async_copy(
            cat_hbm.at[pl.ds(pl.multiple_of(B * K + woff, 8), KP)],
            ind2_v, sem),
    ]
    for cp in cps:
        cp.wait()

    kmax = shift + (K - 1)

    for jc in range(NCHUNK):
        mcomp[pl.ds(jc * L, L)] = (
            2.0e6 + 2.0 * (jc * L + iota).astype(jnp.float32))
    offset = jnp.int32(0)
    for jc in range(NCHUNK):
        sl = pl.ds(jc * L, L)
        kidx = jnp.full((L,), shift + jc * L, jnp.int32) + iota
        if (jc + 1) * L > K:
            kidx = jnp.minimum(kidx, kmax)
        m = plsc.load_gather(mask_v, [kidx])
        mb = m > 0
        if (jc + 1) * L > K:
            mb = mb & (jc * L + iota < K)
            m = jnp.where(mb, m, 0)
        mfv[sl] = m.astype(jnp.float32)
        pos = offset + lax.cumsum(m, axis=0) - 1
        posv[sl] = jnp.where(mb, pos, KP + iota)
        offset = offset + jnp.sum(m)
    n = offset

    for cp in cps2:
        cp.wait()
    psum = jnp.zeros((L,), jnp.float32)
    for jc in range(NCHUNK):
        sl = pl.ds(jc * L, L)
        kidx = jnp.full((L,), shift + jc * L, jnp.int32) + iota
        if (jc + 1) * L > K:
            kidx = jnp.minimum(kidx, kmax)
        ii0 = plsc.load_gather(ind_v, [kidx])
        ii1 = plsc.load_gather(ind2_v, [kidx])
        t0 = plsc.load_gather(tag_a, [ii0 >> 7, ii0 & 127])
        t1 = plsc.load_gather(tag_b, [ii1 >> 7, ii1 & 127])
        mean = (t0 + t1) * 0.5
        d = t0 - t1
        psum = psum + mfv[sl] * (d * d)
        plsc.store_scatter(mcomp, [posv[sl]], mean)
    pullsq = jnp.sum(psum)

    nb = (n + L - 1) >> 4
    my_blocks = (nb - h + 1) >> 1
    zero = jnp.zeros((L,), jnp.float32)

    def outer(t, accs):
        rb = 2 * t + h
        base = rb * L
        basev = jnp.full((L,), base, jnp.int32)
        riv = plsc.load_gather(mcomp, [basev + iota])
        valid = (basev + iota) < n
        mis = [plsc.load_gather(mcomp, [jnp.full((L,), base + l, jnp.int32)])
               for l in range(L)]
        accl = list(accs)
        for l in range(L):
            r = jnp.maximum(1.0 - jnp.abs(riv - mis[l]), 0.0)
            accl[l % NACC] = accl[l % NACC] + jnp.where(
                (iota >= l) & valid, r, 0.0)

        def inner(jc, accs2):
            mjv = plsc.load_gather(
                mcomp, [jnp.full((L,), jc * L, jnp.int32) + iota])
            a2 = list(accs2)
            for l in range(L):
                r = jnp.maximum(1.0 - jnp.abs(mjv - mis[l]), 0.0)
                a2[l % NACC] = a2[l % NACC] + r
            return tuple(a2)

        return lax.fori_loop(rb + 1, nb, inner, tuple(accl))

    accs = lax.fori_loop(0, my_blocks, outer, (zero,) * NACC)
    Sp = jnp.sum(sum(accs[1:], accs[0]))

    h0 = h == 0
    nfv = jnp.full((L,), n.astype(jnp.float32))
    pullv = jnp.full((L,), jnp.where(h0, pullsq, 0.0)) / (2.0 * (nfv + 1e-4))
    corr = jnp.where(h0, nfv + nfv * nfv / (nfv + 1e-4), 0.0)
    pushv = (2.0 * jnp.full((L,), Sp) - corr) / ((nfv - 1.0) * nfv + 1e-4)
    lane = lax.iota(jnp.int32, L)
    row_v[...] = jnp.where(lane == 0, pullv,
                           jnp.where(lane == 1, pushv, 0.0))
    pltpu.sync_copy(row_v, out_hbm.at[c * NS + s])


@functools.partial(
    pl.kernel,
    out_type=jax.ShapeDtypeStruct((NC * NS, L), jnp.float32),
    mesh=plsc.VectorSubcoreMesh(core_axis_name="c", subcore_axis_name="s"),
    compiler_params=pltpu.CompilerParams(needs_layout_passes=False),
    scratch_types=[
        pltpu.VMEM((128, 128), jnp.float32),
        pltpu.VMEM((128, 128), jnp.float32),
        pltpu.VMEM((KP,), jnp.int32),
        pltpu.VMEM((KP,), jnp.int32),
        pltpu.VMEM((KP,), jnp.int32),
        pltpu.VMEM((KP,), jnp.int32),
        pltpu.VMEM((KP,), jnp.float32),
        pltpu.VMEM((KP + L,), jnp.float32),
        pltpu.VMEM((L,), jnp.float32),
        pltpu.SemaphoreType.DMA,
        pltpu.SemaphoreType.DMA,
    ],
)
def _tag_loss_sc(tag1_hbm, tag2_hbm, cat_hbm, out_hbm, *scratch):
    _tec_body(tag1_hbm, tag2_hbm, cat_hbm, out_hbm, *scratch)


@jax.jit
def kernel(tag1, tag2, ind1, ind2, mask):
    cat = jnp.concatenate(
        [ind1.astype(jnp.int32), ind2.astype(jnp.int32),
         mask.astype(jnp.int32)], axis=0).reshape(3 * B * K)
    out = _tag_loss_sc(tag1, tag2, cat)
    return (out[:, 0].sum(), out[:, 1].sum())


# speedup vs baseline: 1.2120x; 1.0500x over previous
import functools

import jax
import jax.numpy as jnp
from jax import lax
from jax.experimental import pallas as pl
from jax.experimental.pallas import tpu as pltpu
from jax.experimental.pallas import tpu_sc as plsc

NC, NS, L = 2, 16, 16
B = 16
K = 500
KP = 512
NCHUNK = KP // L
HW = 128 * 128
NACC = 4


def _tec_body(tag1_hbm, tag2_hbm, cat_hbm, out_hbm,
              tag_a, tag_b, ind_v, ind2_v, mask_v, posv, mfv, mcomp,
              row_v, sem, sem2):
    c = lax.axis_index("c")
    s = lax.axis_index("s")
    b = c * (B // NC) + (s >> 1)
    h = s & 1
    iota = lax.iota(jnp.int32, L)

    bk = b * K
    woff = jnp.minimum(bk & ~7, B * K - KP)
    shift = bk - woff
    mask_off = pl.multiple_of(2 * (B * K) + woff, 8)
    cps2 = [
        pltpu.async_copy(tag1_hbm.at[b, 0], tag_a, sem2),
        pltpu.async_copy(tag2_hbm.at[b, 0], tag_b, sem2),
    ]
    cps = [
        pltpu.async_copy(cat_hbm.at[pl.ds(mask_off, KP)], mask_v, sem),
        pltpu.async_copy(
            cat_hbm.at[pl.ds(pl.multiple_of(woff, 8), KP)], ind_v, sem),
        pltpu.async_copy(
            cat_hbm.at[pl.ds(pl.multiple_of(B * K + woff, 8), KP)],
            ind2_v, sem),
    ]
    for cp in cps:
        cp.wait()

    kmax = shift + (K - 1)

    def mask_body(jc, offset):
        slot = jc * L + iota
        plsc.store_scatter(
            mcomp, [slot], 2.0e6 + 2.0 * slot.astype(jnp.float32))
        kidx = jnp.minimum(jnp.full((L,), shift, jnp.int32) + slot, kmax)
        m = plsc.load_gather(mask_v, [kidx])
        mb = (m > 0) & (slot < K)
        m = jnp.where(mb, m, 0)
        plsc.store_scatter(mfv, [slot], m.astype(jnp.float32))
        pos = offset + lax.cumsum(m, axis=0) - 1
        plsc.store_scatter(posv, [slot], jnp.where(mb, pos, KP + iota))
        return offset + jnp.sum(m)

    n = lax.fori_loop(0, NCHUNK, mask_body, jnp.int32(0))

    for cp in cps2:
        cp.wait()
    psum = jnp.zeros((L,), jnp.float32)
    for jc in range(NCHUNK):
        sl = pl.ds(jc * L, L)
        kidx = jnp.full((L,), shift + jc * L, jnp.int32) + iota
        if (jc + 1) * L > K:
            kidx = jnp.minimum(kidx, kmax)
        ii0 = plsc.load_gather(ind_v, [kidx])
        ii1 = plsc.load_gather(ind2_v, [kidx])
        t0 = plsc.load_gather(tag_a, [ii0 >> 7, ii0 & 127])
        t1 = plsc.load_gather(tag_b, [ii1 >> 7, ii1 & 127])
        mean = (t0 + t1) * 0.5
        d = t0 - t1
        psum = psum + mfv[sl] * (d * d)
        plsc.store_scatter(mcomp, [posv[sl]], mean)
    pullsq = jnp.sum(psum)

    nb = (n + L - 1) >> 4
    my_blocks = (nb - h + 1) >> 1
    zero = jnp.zeros((L,), jnp.float32)

    def outer(t, accs):
        rb = 2 * t + h
        base = rb * L
        basev = jnp.full((L,), base, jnp.int32)
        riv = plsc.load_gather(mcomp, [basev + iota])
        valid = (basev + iota) < n
        mis = [plsc.load_gather(mcomp, [jnp.full((L,), base + l, jnp.int32)])
               for l in range(L)]
        accl = list(accs)
        for l in range(L):
            r = jnp.maximum(1.0 - jnp.abs(riv - mis[l]), 0.0)
            accl[l % NACC] = accl[l % NACC] + jnp.where(
                (iota >= l) & valid, r, 0.0)

        def inner(jc, accs2):
            mjv = plsc.load_gather(
                mcomp, [jnp.full((L,), jc * L, jnp.int32) + iota])
            a2 = list(accs2)
            for l in range(L):
                r = jnp.maximum(1.0 - jnp.abs(mjv - mis[l]), 0.0)
                a2[l % NACC] = a2[l % NACC] + r
            return tuple(a2)

        return lax.fori_loop(rb + 1, nb, inner, tuple(accl))

    accs = lax.fori_loop(0, my_blocks, outer, (zero,) * NACC)
    Sp = jnp.sum(sum(accs[1:], accs[0]))

    h0 = h == 0
    nfv = jnp.full((L,), n.astype(jnp.float32))
    pullv = jnp.full((L,), jnp.where(h0, pullsq, 0.0)) / (2.0 * (nfv + 1e-4))
    corr = jnp.where(h0, nfv + nfv * nfv / (nfv + 1e-4), 0.0)
    pushv = (2.0 * jnp.full((L,), Sp) - corr) / ((nfv - 1.0) * nfv + 1e-4)
    lane = lax.iota(jnp.int32, L)
    row_v[...] = jnp.where(lane == 0, pullv,
                           jnp.where(lane == 1, pushv, 0.0))
    pltpu.sync_copy(row_v, out_hbm.at[c * NS + s])


@functools.partial(
    pl.kernel,
    out_type=jax.ShapeDtypeStruct((NC * NS, L), jnp.float32),
    mesh=plsc.VectorSubcoreMesh(core_axis_name="c", subcore_axis_name="s"),
    compiler_params=pltpu.CompilerParams(needs_layout_passes=False),
    scratch_types=[
        pltpu.VMEM((128, 128), jnp.float32),
        pltpu.VMEM((128, 128), jnp.float32),
        pltpu.VMEM((KP,), jnp.int32),
        pltpu.VMEM((KP,), jnp.int32),
        pltpu.VMEM((KP,), jnp.int32),
        pltpu.VMEM((KP,), jnp.int32),
        pltpu.VMEM((KP,), jnp.float32),
        pltpu.VMEM((KP + L,), jnp.float32),
        pltpu.VMEM((L,), jnp.float32),
        pltpu.SemaphoreType.DMA,
        pltpu.SemaphoreType.DMA,
    ],
)
def _tag_loss_sc(tag1_hbm, tag2_hbm, cat_hbm, out_hbm, *scratch):
    _tec_body(tag1_hbm, tag2_hbm, cat_hbm, out_hbm, *scratch)


@jax.jit
def kernel(tag1, tag2, ind1, ind2, mask):
    cat = jnp.concatenate(
        [ind1.astype(jnp.int32), ind2.astype(jnp.int32),
         mask.astype(jnp.int32)], axis=0).reshape(3 * B * K)
    out = _tag_loss_sc(tag1, tag2, cat)
    return (out[:, 0].sum(), out[:, 1].sum())


# speedup vs baseline: 1.2542x; 1.0348x over previous
import functools

import jax
import jax.numpy as jnp
from jax import lax
from jax.experimental import pallas as pl
from jax.experimental.pallas import tpu as pltpu
from jax.experimental.pallas import tpu_sc as plsc

NC, NS, L = 2, 16, 16
B = 16
K = 500
KP = 512
NCHUNK = KP // L
HW = 128 * 128
NACC = 4


def _tec_body(tag1_hbm, tag2_hbm, cat_hbm, out_hbm,
              tag_a, tag_b, ind_v, ind2_v, mask_v, posv, mfv, mcomp,
              row_v, sem, sem2):
    c = lax.axis_index("c")
    s = lax.axis_index("s")
    b = c * (B // NC) + (s >> 1)
    h = s & 1
    iota = lax.iota(jnp.int32, L)

    bk = b * K
    woff = jnp.minimum(bk & ~7, B * K - KP)
    shift = bk - woff
    mask_off = pl.multiple_of(2 * (B * K) + woff, 8)
    cps2 = [
        pltpu.async_copy(tag1_hbm.at[b, 0], tag_a, sem2),
        pltpu.async_copy(tag2_hbm.at[b, 0], tag_b, sem2),
    ]
    cps = [
        pltpu.async_copy(cat_hbm.at[pl.ds(mask_off, KP)], mask_v, sem),
        pltpu.async_copy(
            cat_hbm.at[pl.ds(pl.multiple_of(woff, 8), KP)], ind_v, sem),
        pltpu.async_copy(
            cat_hbm.at[pl.ds(pl.multiple_of(B * K + woff, 8), KP)],
            ind2_v, sem),
    ]
    for cp in cps:
        cp.wait()

    kmax = shift + (K - 1)

    def mask_body(jc, offset):
        slot = jc * L + iota
        plsc.store_scatter(
            mcomp, [slot], 2.0e6 + 2.0 * slot.astype(jnp.float32))
        kidx = jnp.minimum(jnp.full((L,), shift, jnp.int32) + slot, kmax)
        m = plsc.load_gather(mask_v, [kidx])
        mb = (m > 0) & (slot < K)
        m = jnp.where(mb, m, 0)
        plsc.store_scatter(mfv, [slot], m.astype(jnp.float32))
        pos = offset + lax.cumsum(m, axis=0) - 1
        plsc.store_scatter(posv, [slot], jnp.where(mb, pos, KP + iota))
        return offset + jnp.sum(m)

    n = lax.fori_loop(0, NCHUNK, mask_body, jnp.int32(0))

    for cp in cps2:
        cp.wait()

    def gather_body(jc, psum):
        slot = jc * L + iota
        kidx = jnp.minimum(jnp.full((L,), shift, jnp.int32) + slot, kmax)
        ii0 = plsc.load_gather(ind_v, [kidx])
        ii1 = plsc.load_gather(ind2_v, [kidx])
        t0 = plsc.load_gather(tag_a, [ii0 >> 7, ii0 & 127])
        t1 = plsc.load_gather(tag_b, [ii1 >> 7, ii1 & 127])
        mean = (t0 + t1) * 0.5
        d = t0 - t1
        plsc.store_scatter(mcomp, [plsc.load_gather(posv, [slot])], mean)
        return psum + plsc.load_gather(mfv, [slot]) * (d * d)

    psum = lax.fori_loop(0, NCHUNK, gather_body,
                         jnp.zeros((L,), jnp.float32))
    pullsq = jnp.sum(psum)

    nb = (n + L - 1) >> 4
    my_blocks = (nb - h + 1) >> 1
    zero = jnp.zeros((L,), jnp.float32)

    def outer(t, accs):
        rb = 2 * t + h
        base = rb * L
        basev = jnp.full((L,), base, jnp.int32)
        riv = plsc.load_gather(mcomp, [basev + iota])
        valid = (basev + iota) < n
        mis = [plsc.load_gather(mcomp, [jnp.full((L,), base + l, jnp.int32)])
               for l in range(L)]
        accl = list(accs)
        for l in range(L):
            r = jnp.maximum(1.0 - jnp.abs(riv - mis[l]), 0.0)
            accl[l % NACC] = accl[l % NACC] + jnp.where(
                (iota >= l) & valid, r, 0.0)

        def inner(jc, accs2):
            mjv = plsc.load_gather(
                mcomp, [jnp.full((L,), jc * L, jnp.int32) + iota])
            a2 = list(accs2)
            for l in range(L):
                r = jnp.maximum(1.0 - jnp.abs(mjv - mis[l]), 0.0)
                a2[l % NACC] = a2[l % NACC] + r
            return tuple(a2)

        return lax.fori_loop(rb + 1, nb, inner, tuple(accl))

    accs = lax.fori_loop(0, my_blocks, outer, (zero,) * NACC)
    Sp = jnp.sum(sum(accs[1:], accs[0]))

    h0 = h == 0
    nfv = jnp.full((L,), n.astype(jnp.float32))
    pullv = jnp.full((L,), jnp.where(h0, pullsq, 0.0)) / (2.0 * (nfv + 1e-4))
    corr = jnp.where(h0, nfv + nfv * nfv / (nfv + 1e-4), 0.0)
    pushv = (2.0 * jnp.full((L,), Sp) - corr) / ((nfv - 1.0) * nfv + 1e-4)
    lane = lax.iota(jnp.int32, L)
    row_v[...] = jnp.where(lane == 0, pullv,
                           jnp.where(lane == 1, pushv, 0.0))
    pltpu.sync_copy(row_v, out_hbm.at[c * NS + s])


@functools.partial(
    pl.kernel,
    out_type=jax.ShapeDtypeStruct((NC * NS, L), jnp.float32),
    mesh=plsc.VectorSubcoreMesh(core_axis_name="c", subcore_axis_name="s"),
    compiler_params=pltpu.CompilerParams(needs_layout_passes=False),
    scratch_types=[
        pltpu.VMEM((128, 128), jnp.float32),
        pltpu.VMEM((128, 128), jnp.float32),
        pltpu.VMEM((KP,), jnp.int32),
        pltpu.VMEM((KP,), jnp.int32),
        pltpu.VMEM((KP,), jnp.int32),
        pltpu.VMEM((KP,), jnp.int32),
        pltpu.VMEM((KP,), jnp.float32),
        pltpu.VMEM((KP + L,), jnp.float32),
        pltpu.VMEM((L,), jnp.float32),
        pltpu.SemaphoreType.DMA,
        pltpu.SemaphoreType.DMA,
    ],
)
def _tag_loss_sc(tag1_hbm, tag2_hbm, cat_hbm, out_hbm, *scratch):
    _tec_body(tag1_hbm, tag2_hbm, cat_hbm, out_hbm, *scratch)


@jax.jit
def kernel(tag1, tag2, ind1, ind2, mask):
    cat = jnp.concatenate(
        [ind1.astype(jnp.int32), ind2.astype(jnp.int32),
         mask.astype(jnp.int32)], axis=0).reshape(3 * B * K)
    out = _tag_loss_sc(tag1, tag2, cat)
    return (out[:, 0].sum(), out[:, 1].sum())


# speedup vs baseline: 1.2698x; 1.0124x over previous
import functools

import jax
import jax.numpy as jnp
from jax import lax
from jax.experimental import pallas as pl
from jax.experimental.pallas import tpu as pltpu
from jax.experimental.pallas import tpu_sc as plsc

NC, NS, L = 2, 16, 16
B = 16
K = 500
KP = 512
NCHUNK = KP // L
HW = 128 * 128
NACC = 4


def _tec_body(tag1_hbm, tag2_hbm, cat_hbm, out_hbm,
              tag_a, ind_v, mask_v, posv, mfv, tmine, tother, mcomp,
              row_v, shared, sem, sem2):
    c = lax.axis_index("c")
    s = lax.axis_index("s")
    b = c * (B // NC) + (s >> 1)
    h = s & 1
    iota = lax.iota(jnp.int32, L)

    bk = b * K
    woff = jnp.minimum(bk & ~7, B * K - KP)
    shift = bk - woff
    mask_off = pl.multiple_of(2 * (B * K) + woff, 8)
    pltpu.async_copy(cat_hbm.at[pl.ds(mask_off, KP)], mask_v, sem)
    for sid in range(NS):

        @pl.when(s == sid)
        def _(sid=sid):
            if sid % 2 == 0:
                pltpu.async_copy(tag1_hbm.at[b, 0], tag_a, sem2)
                pltpu.async_copy(
                    cat_hbm.at[pl.ds(pl.multiple_of(woff, 8), KP)],
                    ind_v, sem)
            else:
                pltpu.async_copy(tag2_hbm.at[b, 0], tag_a, sem2)
                pltpu.async_copy(
                    cat_hbm.at[pl.ds(pl.multiple_of(B * K + woff, 8), KP)],
                    ind_v, sem)

    pltpu.make_async_copy(cat_hbm.at[pl.ds(mask_off, KP)], mask_v,
                          sem).wait()
    pltpu.make_async_copy(cat_hbm.at[pl.ds(mask_off, KP)], ind_v,
                          sem).wait()

    kmax = shift + (K - 1)

    def mask_body(jc, offset):
        slot = jc * L + iota
        plsc.store_scatter(
            mcomp, [slot], 2.0e6 + 2.0 * slot.astype(jnp.float32))
        kidx = jnp.minimum(jnp.full((L,), shift, jnp.int32) + slot, kmax)
        m = plsc.load_gather(mask_v, [kidx])
        mb = (m > 0) & (slot < K)
        m = jnp.where(mb, m, 0)
        plsc.store_scatter(mfv, [slot], m.astype(jnp.float32))
        pos = offset + lax.cumsum(m, axis=0) - 1
        plsc.store_scatter(posv, [slot], jnp.where(mb, pos, KP + iota))
        return offset + jnp.sum(m)

    n = lax.fori_loop(0, NCHUNK, mask_body, jnp.int32(0))

    pltpu.make_async_copy(tag1_hbm.at[b, 0], tag_a, sem2).wait()

    def gather_body(jc, carry):
        slot = jc * L + iota
        kidx = jnp.minimum(jnp.full((L,), shift, jnp.int32) + slot, kmax)
        ii = plsc.load_gather(ind_v, [kidx])
        plsc.store_scatter(
            tmine, [slot], plsc.load_gather(tag_a, [ii >> 7, ii & 127]))
        return carry

    lax.fori_loop(0, NCHUNK, gather_body, jnp.int32(0))
    pltpu.sync_copy(tmine, shared.at[s])
    plsc.subcore_barrier()
    pltpu.sync_copy(shared.at[s ^ 1], tother)

    def combine_body(jc, psum):
        slot = jc * L + iota
        t0 = plsc.load_gather(tmine, [slot])
        t1 = plsc.load_gather(tother, [slot])
        mean = (t0 + t1) * 0.5
        d = t0 - t1
        plsc.store_scatter(mcomp, [plsc.load_gather(posv, [slot])], mean)
        return psum + plsc.load_gather(mfv, [slot]) * (d * d)

    psum = lax.fori_loop(0, NCHUNK, combine_body,
                         jnp.zeros((L,), jnp.float32))
    pullsq = jnp.sum(psum)

    nb = (n + L - 1) >> 4
    my_blocks = (nb - h + 1) >> 1
    zero = jnp.zeros((L,), jnp.float32)

    def outer(t, accs):
        rb = 2 * t + h
        base = rb * L
        basev = jnp.full((L,), base, jnp.int32)
        riv = plsc.load_gather(mcomp, [basev + iota])
        valid = (basev + iota) < n
        mis = [plsc.load_gather(mcomp, [jnp.full((L,), base + l, jnp.int32)])
               for l in range(L)]
        accl = list(accs)
        for l in range(L):
            r = jnp.maximum(1.0 - jnp.abs(riv - mis[l]), 0.0)
            accl[l % NACC] = accl[l % NACC] + jnp.where(
                (iota >= l) & valid, r, 0.0)

        def inner(jc, accs2):
            mjv = plsc.load_gather(
                mcomp, [jnp.full((L,), jc * L, jnp.int32) + iota])
            a2 = list(accs2)
            for l in range(L):
                r = jnp.maximum(1.0 - jnp.abs(mjv - mis[l]), 0.0)
                a2[l % NACC] = a2[l % NACC] + r
            return tuple(a2)

        return lax.fori_loop(rb + 1, nb, inner, tuple(accl))

    accs = lax.fori_loop(0, my_blocks, outer, (zero,) * NACC)
    Sp = jnp.sum(sum(accs[1:], accs[0]))

    h0 = h == 0
    nfv = jnp.full((L,), n.astype(jnp.float32))
    pullv = jnp.full((L,), jnp.where(h0, pullsq, 0.0)) / (2.0 * (nfv + 1e-4))
    corr = jnp.where(h0, nfv + nfv * nfv / (nfv + 1e-4), 0.0)
    pushv = (2.0 * jnp.full((L,), Sp) - corr) / ((nfv - 1.0) * nfv + 1e-4)
    lane = lax.iota(jnp.int32, L)
    row_v[...] = jnp.where(lane == 0, pullv,
                           jnp.where(lane == 1, pushv, 0.0))
    pltpu.sync_copy(row_v, out_hbm.at[c * NS + s])


@functools.partial(
    pl.kernel,
    out_type=jax.ShapeDtypeStruct((NC * NS, L), jnp.float32),
    mesh=plsc.VectorSubcoreMesh(core_axis_name="c", subcore_axis_name="s"),
    compiler_params=pltpu.CompilerParams(needs_layout_passes=False),
    scratch_types=[
        pltpu.VMEM((128, 128), jnp.float32),
        pltpu.VMEM((KP,), jnp.int32),
        pltpu.VMEM((KP,), jnp.int32),
        pltpu.VMEM((KP,), jnp.int32),
        pltpu.VMEM((KP,), jnp.float32),
        pltpu.VMEM((KP,), jnp.float32),
        pltpu.VMEM((KP,), jnp.float32),
        pltpu.VMEM((KP + L,), jnp.float32),
        pltpu.VMEM((L,), jnp.float32),
        pltpu.VMEM_SHARED((NS, KP), jnp.float32),
        pltpu.SemaphoreType.DMA,
        pltpu.SemaphoreType.DMA,
    ],
)
def _tag_loss_sc(tag1_hbm, tag2_hbm, cat_hbm, out_hbm, *scratch):
    _tec_body(tag1_hbm, tag2_hbm, cat_hbm, out_hbm, *scratch)


@jax.jit
def kernel(tag1, tag2, ind1, ind2, mask):
    cat = jnp.concatenate(
        [ind1.astype(jnp.int32), ind2.astype(jnp.int32),
         mask.astype(jnp.int32)], axis=0).reshape(3 * B * K)
    out = _tag_loss_sc(tag1, tag2, cat)
    return (out[:, 0].sum(), out[:, 1].sum())
